# baseline (speedup 1.0000x reference)
import functools

import jax
import jax.numpy as jnp
from jax import lax
from jax.experimental import pallas as pl
from jax.experimental.pallas import tpu as pltpu
from jax.experimental.pallas import tpu_sc as plsc

N = 4096
NC = 2
NS = 16
NW = NC * NS
L = 16
RPW = N // NW
GROUP = 16
NCHUNK = N // L


def _prologue_body(sim_ref, p_ref, hp_ref, iou_ref, hn_ref, w_ref, qn_ref,
                   prm_ref):
    p = p_ref[...]
    hp = hp_ref[...]
    iou = iou_ref[...]
    sim = sim_ref[...]
    hn = p + hp - 2.0 * p * hp
    S = 1.0 + jnp.sum(hn * iou)
    logS = jnp.log(S)
    w = jnp.where(hn > 0.0, jnp.log(iou) - logS, 0.0)
    qn = jnp.exp(sim) * hp
    T = jnp.sum(qn)
    cnt = jnp.sum(hp)
    hn_ref[...] = hn
    w_ref[...] = w
    qn_ref[...] = qn
    z = jnp.float32(0.0)
    vals = jnp.stack([-logS, T, cnt, z, z, z, z, z])
    prm_ref[...] = jnp.broadcast_to(vals[:, None], (8, 128))


_prologue = pl.pallas_call(
    _prologue_body,
    out_shape=(
        jax.ShapeDtypeStruct((32, 128), jnp.float32),
        jax.ShapeDtypeStruct((32, 128), jnp.float32),
        jax.ShapeDtypeStruct((32, 128), jnp.float32),
        jax.ShapeDtypeStruct((8, 128), jnp.float32),
    ),
)


def _sc_body(d_hbm, hn_hbm, w_hbm, qn_hbm, hp_hbm, prm_hbm, out_hbm,
             hn_v, w_v, rows_v, mlogs_v, t_v, qn16_v, hp16_v, tot_v):
    c = lax.axis_index("c")
    s = lax.axis_index("s")
    wid = s * NC + c
    row0 = wid * RPW

    pltpu.sync_copy(hn_hbm, hn_v)
    pltpu.sync_copy(w_hbm, w_v)
    pltpu.sync_copy(prm_hbm.at[pl.ds(0, L)], mlogs_v)
    pltpu.sync_copy(prm_hbm.at[pl.ds(128, L)], t_v)
    mlogs = mlogs_v[...]
    t16 = t_v[...]

    iota = lax.iota(jnp.int32, L)
    total = jnp.zeros((L,), jnp.float32)

    for g in range(RPW // GROUP):
        r0 = row0 + g * GROUP
        pltpu.sync_copy(d_hbm.at[pl.ds(r0 * N, GROUP * N)], rows_v)

        def row_body(r, carry):
            ra, rb = carry
            rbase = r * N

            def chunk(j, inner):
                a, b = inner
                e = jnp.exp(rows_v[pl.ds(rbase + j * L, L)])
                a = a + e * hn_v[pl.ds(j * L, L)]
                b = b + e * w_v[pl.ds(j * L, L)]
                return (a, b)

            acc_a, acc_b = lax.fori_loop(
                0, NCHUNK, chunk,
                (jnp.zeros((L,), jnp.float32), jnp.zeros((L,), jnp.float32)))
            sel = iota == r
            ra = jnp.where(sel, jnp.full((L,), jnp.sum(acc_a)), ra)
            rb = jnp.where(sel, jnp.full((L,), jnp.sum(acc_b)), rb)
            return (ra, rb)

        ra, rb = lax.fori_loop(
            0, GROUP, row_body,
            (jnp.zeros((L,), jnp.float32), jnp.zeros((L,), jnp.float32)))

        pltpu.sync_copy(qn_hbm.at[pl.ds(r0, L)], qn16_v)
        pltpu.sync_copy(hp_hbm.at[pl.ds(r0, L)], hp16_v)
        qn16 = qn16_v[...]
        hp16 = hp16_v[...]
        total = total + hp16 * (qn16 * mlogs + rb) / (ra + t16)

    tot_v[...] = total
    pltpu.sync_copy(tot_v, out_hbm.at[pl.ds(wid * L, L)])


_sc_main = functools.partial(
    pl.kernel,
    out_type=jax.ShapeDtypeStruct((NW * L,), jnp.float32),
    mesh=plsc.VectorSubcoreMesh(core_axis_name="c", subcore_axis_name="s"),
    compiler_params=pltpu.CompilerParams(needs_layout_passes=False),
    scratch_types=[
        pltpu.VMEM((N,), jnp.float32),
        pltpu.VMEM((N,), jnp.float32),
        pltpu.VMEM((GROUP * N,), jnp.float32),
        pltpu.VMEM((L,), jnp.float32),
        pltpu.VMEM((L,), jnp.float32),
        pltpu.VMEM((L,), jnp.float32),
        pltpu.VMEM((L,), jnp.float32),
        pltpu.VMEM((L,), jnp.float32),
    ],
)(_sc_body)


def _epilogue_body(part_ref, prm_ref, out_ref):
    loss = -jnp.sum(part_ref[...])
    cnt = prm_ref[...][2, 0]
    out_ref[...] = jnp.where(cnt == 0.0, 0.0, loss)[None, None]


_epilogue = pl.pallas_call(
    _epilogue_body,
    out_shape=jax.ShapeDtypeStruct((1, 1), jnp.float32),
)


def kernel(sim_mat, database_sim_mat, pos_mask, hard_pos_mask, neg_mask, iou):
    del neg_mask
    p_f = pos_mask[:, 1].astype(jnp.float32)
    hp_f = hard_pos_mask[:, 1].astype(jnp.float32)
    iou_ = iou[:, 0]

    hn, w, qn, prm = _prologue(
        sim_mat.reshape(32, 128), p_f.reshape(32, 128),
        hp_f.reshape(32, 128), iou_.reshape(32, 128))

    partials = _sc_main(
        database_sim_mat.reshape(N * N),
        hn.reshape(N), w.reshape(N), qn.reshape(N), hp_f,
        prm.reshape(8 * 128))

    out = _epilogue(partials.reshape(4, 128), prm)
    return out.reshape(())


# speedup vs baseline: 1.7978x; 1.7978x over previous
import functools

import jax
import jax.numpy as jnp
from jax import lax
from jax.experimental import pallas as pl
from jax.experimental.pallas import tpu as pltpu
from jax.experimental.pallas import tpu_sc as plsc

N = 4096
NC = 2
NS = 16
NW = NC * NS
L = 16
RPW = N // NW
GROUP = 16
NCHUNK = N // L


def _prologue_body(sim_ref, p_ref, hp_ref, iou_ref, hn_ref, w_ref, qn_ref,
                   prm_ref):
    p = p_ref[...]
    hp = hp_ref[...]
    iou = iou_ref[...]
    sim = sim_ref[...]
    hn = p + hp - 2.0 * p * hp
    S = 1.0 + jnp.sum(hn * iou)
    logS = jnp.log(S)
    w = jnp.where(hn > 0.0, jnp.log(iou) - logS, 0.0)
    qn = jnp.exp(sim) * hp
    T = jnp.sum(qn)
    cnt = jnp.sum(hp)
    hn_ref[...] = hn
    w_ref[...] = w
    qn_ref[...] = qn
    z = jnp.float32(0.0)
    vals = jnp.stack([-logS, T, cnt, z, z, z, z, z])
    prm_ref[...] = jnp.broadcast_to(vals[:, None], (8, 128))


_prologue = pl.pallas_call(
    _prologue_body,
    out_shape=(
        jax.ShapeDtypeStruct((32, 128), jnp.float32),
        jax.ShapeDtypeStruct((32, 128), jnp.float32),
        jax.ShapeDtypeStruct((32, 128), jnp.float32),
        jax.ShapeDtypeStruct((8, 128), jnp.float32),
    ),
)


RB = 8
NBLK = RPW // RB


def _sc_body(d_hbm, hn_hbm, w_hbm, qn_hbm, hp_hbm, prm_hbm, out_hbm,
             hn_v, w_v, rows0_v, rows1_v, mlogs_v, t_v, qn16_v, hp16_v,
             tot_v, sem0, sem1):
    c = lax.axis_index("c")
    s = lax.axis_index("s")
    wid = s * NC + c
    row0 = wid * RPW

    pltpu.sync_copy(hn_hbm, hn_v)
    pltpu.sync_copy(w_hbm, w_v)
    pltpu.sync_copy(prm_hbm.at[pl.ds(0, L)], mlogs_v)
    pltpu.sync_copy(prm_hbm.at[pl.ds(128, L)], t_v)
    mlogs = mlogs_v[...]
    t16 = t_v[...]

    iota = lax.iota(jnp.int32, L)
    lo8 = iota < RB
    total = jnp.zeros((L,), jnp.float32)
    zero = jnp.zeros((L,), jnp.float32)

    bufs = (rows0_v, rows1_v)
    sems = (sem0, sem1)

    cp = pltpu.async_copy(d_hbm.at[pl.ds(row0 * N, RB * N)], rows0_v, sem0)
    for blk in range(NBLK):
        buf = bufs[blk % 2]
        r0 = row0 + blk * RB
        if blk + 1 < NBLK:
            cp_next = pltpu.async_copy(
                d_hbm.at[pl.ds((r0 + RB) * N, RB * N)],
                bufs[(blk + 1) % 2], sems[(blk + 1) % 2])
        cp.wait()

        def chunk(j, carry):
            base = j * L
            hn16 = hn_v[pl.ds(base, L)]
            w16 = w_v[pl.ds(base, L)]
            outs = []
            for r in range(RB):
                e = jnp.exp(buf[pl.ds(r * N + base, L)])
                outs.append(carry[2 * r] + e * hn16)
                outs.append(carry[2 * r + 1] + e * w16)
            return tuple(outs)

        accs = lax.fori_loop(0, NCHUNK, chunk, (zero,) * (2 * RB))

        ra = zero
        rb = zero
        for r in range(RB):
            sel = iota == r
            ra = jnp.where(sel, jnp.full((L,), jnp.sum(accs[2 * r])), ra)
            rb = jnp.where(sel, jnp.full((L,), jnp.sum(accs[2 * r + 1])), rb)

        pltpu.sync_copy(qn_hbm.at[pl.ds(r0, RB)], qn16_v.at[pl.ds(0, RB)])
        pltpu.sync_copy(hp_hbm.at[pl.ds(r0, RB)], hp16_v.at[pl.ds(0, RB)])
        qn16 = qn16_v[...]
        hp16 = hp16_v[...]
        contrib = hp16 * (qn16 * mlogs + rb) / (ra + t16)
        total = total + jnp.where(lo8, contrib, zero)
        if blk + 1 < NBLK:
            cp = cp_next

    tot_v[...] = total
    pltpu.sync_copy(tot_v, out_hbm.at[pl.ds(wid * L, L)])


_sc_main = functools.partial(
    pl.kernel,
    out_type=jax.ShapeDtypeStruct((NW * L,), jnp.float32),
    mesh=plsc.VectorSubcoreMesh(core_axis_name="c", subcore_axis_name="s"),
    compiler_params=pltpu.CompilerParams(needs_layout_passes=False),
    scratch_types=[
        pltpu.VMEM((N,), jnp.float32),
        pltpu.VMEM((N,), jnp.float32),
        pltpu.VMEM((RB * N,), jnp.float32),
        pltpu.VMEM((RB * N,), jnp.float32),
        pltpu.VMEM((L,), jnp.float32),
        pltpu.VMEM((L,), jnp.float32),
        pltpu.VMEM((L,), jnp.float32),
        pltpu.VMEM((L,), jnp.float32),
        pltpu.VMEM((L,), jnp.float32),
        pltpu.SemaphoreType.DMA,
        pltpu.SemaphoreType.DMA,
    ],
)(_sc_body)


def _epilogue_body(part_ref, prm_ref, out_ref):
    loss = -jnp.sum(part_ref[...])
    cnt = prm_ref[...][2, 0]
    out_ref[...] = jnp.where(cnt == 0.0, 0.0, loss)[None, None]


_epilogue = pl.pallas_call(
    _epilogue_body,
    out_shape=jax.ShapeDtypeStruct((1, 1), jnp.float32),
)


def kernel(sim_mat, database_sim_mat, pos_mask, hard_pos_mask, neg_mask, iou):
    del neg_mask
    p_f = pos_mask[:, 1].astype(jnp.float32)
    hp_f = hard_pos_mask[:, 1].astype(jnp.float32)
    iou_ = iou[:, 0]

    hn, w, qn, prm = _prologue(
        sim_mat.reshape(32, 128), p_f.reshape(32, 128),
        hp_f.reshape(32, 128), iou_.reshape(32, 128))

    partials = _sc_main(
        database_sim_mat.reshape(N * N),
        hn.reshape(N), w.reshape(N), qn.reshape(N), hp_f,
        prm.reshape(8 * 128))

    out = _epilogue(partials.reshape(4, 128), prm)
    return out.reshape(())


# speedup vs baseline: 1.9731x; 1.0975x over previous
import functools

import jax
import jax.numpy as jnp
from jax import lax
from jax.experimental import pallas as pl
from jax.experimental.pallas import tpu as pltpu
from jax.experimental.pallas import tpu_sc as plsc

N = 4096
NC = 2
NS = 16
NW = NC * NS
L = 16
RPW = N // NW
GROUP = 16
NCHUNK = N // L


def _prologue_body(sim_ref, p_ref, hp_ref, iou_ref, hn_ref, w_ref, qn_ref,
                   prm_ref):
    p = p_ref[...]
    hp = hp_ref[...]
    iou = iou_ref[...]
    sim = sim_ref[...]
    hn = p + hp - 2.0 * p * hp
    S = 1.0 + jnp.sum(hn * iou)
    logS = jnp.log(S)
    w = jnp.where(hn > 0.0, jnp.log(iou) - logS, 0.0)
    qn = jnp.exp(sim) * hp
    T = jnp.sum(qn)
    cnt = jnp.sum(hp)
    hn_ref[...] = hn
    w_ref[...] = w
    qn_ref[...] = qn
    z = jnp.float32(0.0)
    vals = jnp.stack([-logS, T, cnt, z, z, z, z, z])
    prm_ref[...] = jnp.broadcast_to(vals[:, None], (8, 128))


_prologue = pl.pallas_call(
    _prologue_body,
    out_shape=(
        jax.ShapeDtypeStruct((32, 128), jnp.float32),
        jax.ShapeDtypeStruct((32, 128), jnp.float32),
        jax.ShapeDtypeStruct((32, 128), jnp.float32),
        jax.ShapeDtypeStruct((8, 128), jnp.float32),
    ),
)


RB = 8
NBLK = RPW // RB


def _sc_body(d_hbm, hn_hbm, w_hbm, qn_hbm, hp_hbm, prm_hbm, out_hbm,
             hn_v, w_v, rows0_v, rows1_v, mlogs_v, t_v, qn_v, hp_v,
             tot_v, sem0, sem1):
    c = lax.axis_index("c")
    s = lax.axis_index("s")
    wid = s * NC + c
    row0 = wid * RPW

    pltpu.sync_copy(hn_hbm, hn_v)
    pltpu.sync_copy(w_hbm, w_v)
    pltpu.sync_copy(prm_hbm.at[pl.ds(0, L)], mlogs_v)
    pltpu.sync_copy(prm_hbm.at[pl.ds(128, L)], t_v)
    pltpu.sync_copy(qn_hbm.at[pl.ds(row0, RPW)], qn_v.at[pl.ds(0, RPW)])
    pltpu.sync_copy(hp_hbm.at[pl.ds(row0, RPW)], hp_v.at[pl.ds(0, RPW)])
    mlogs = mlogs_v[...]
    t16 = t_v[...]

    iota = lax.iota(jnp.int32, L)
    lo8 = iota < RB
    total = jnp.zeros((L,), jnp.float32)
    zero = jnp.zeros((L,), jnp.float32)

    bufs = (rows0_v, rows1_v)
    sems = (sem0, sem1)

    cp = pltpu.async_copy(d_hbm.at[pl.ds(row0 * N, RB * N)], rows0_v, sem0)
    for blk in range(NBLK):
        buf = bufs[blk % 2]
        r0 = row0 + blk * RB
        if blk + 1 < NBLK:
            cp_next = pltpu.async_copy(
                d_hbm.at[pl.ds((r0 + RB) * N, RB * N)],
                bufs[(blk + 1) % 2], sems[(blk + 1) % 2])
        cp.wait()

        @plsc.parallel_loop(0, NCHUNK * L, step=L, unroll=2,
                            carry=(zero,) * (2 * RB))
        def accs(base, carry):
            hn16 = hn_v[pl.ds(base, L)]
            w16 = w_v[pl.ds(base, L)]
            outs = []
            for r in range(RB):
                e = jnp.exp(buf[pl.ds(r * N + base, L)])
                outs.append(carry[2 * r] + e * hn16)
                outs.append(carry[2 * r + 1] + e * w16)
            return tuple(outs)

        ra = zero
        rb = zero
        for r in range(RB):
            sel = iota == r
            ra = jnp.where(sel, jnp.full((L,), jnp.sum(accs[2 * r])), ra)
            rb = jnp.where(sel, jnp.full((L,), jnp.sum(accs[2 * r + 1])), rb)

        qn16 = qn_v[pl.ds(blk * RB, L)]
        hp16 = hp_v[pl.ds(blk * RB, L)]
        contrib = hp16 * (qn16 * mlogs + rb) / (ra + t16)
        total = total + jnp.where(lo8, contrib, zero)
        if blk + 1 < NBLK:
            cp = cp_next

    tot_v[...] = total
    pltpu.sync_copy(tot_v, out_hbm.at[pl.ds(wid * L, L)])


_sc_main = functools.partial(
    pl.kernel,
    out_type=jax.ShapeDtypeStruct((NW * L,), jnp.float32),
    mesh=plsc.VectorSubcoreMesh(core_axis_name="c", subcore_axis_name="s"),
    compiler_params=pltpu.CompilerParams(needs_layout_passes=False),
    scratch_types=[
        pltpu.VMEM((N,), jnp.float32),
        pltpu.VMEM((N,), jnp.float32),
        pltpu.VMEM((RB * N,), jnp.float32),
        pltpu.VMEM((RB * N,), jnp.float32),
        pltpu.VMEM((L,), jnp.float32),
        pltpu.VMEM((L,), jnp.float32),
        pltpu.VMEM((RPW + L,), jnp.float32),
        pltpu.VMEM((RPW + L,), jnp.float32),
        pltpu.VMEM((L,), jnp.float32),
        pltpu.SemaphoreType.DMA,
        pltpu.SemaphoreType.DMA,
    ],
)(_sc_body)


def _epilogue_body(part_ref, prm_ref, out_ref):
    loss = -jnp.sum(part_ref[...])
    cnt = prm_ref[...][2, 0]
    out_ref[...] = jnp.where(cnt == 0.0, 0.0, loss)[None, None]


_epilogue = pl.pallas_call(
    _epilogue_body,
    out_shape=jax.ShapeDtypeStruct((1, 1), jnp.float32),
)


def kernel(sim_mat, database_sim_mat, pos_mask, hard_pos_mask, neg_mask, iou):
    del neg_mask
    p_f = pos_mask[:, 1].astype(jnp.float32)
    hp_f = hard_pos_mask[:, 1].astype(jnp.float32)
    iou_ = iou[:, 0]

    hn, w, qn, prm = _prologue(
        sim_mat.reshape(32, 128), p_f.reshape(32, 128),
        hp_f.reshape(32, 128), iou_.reshape(32, 128))

    partials = _sc_main(
        database_sim_mat.reshape(N * N),
        hn.reshape(N), w.reshape(N), qn.reshape(N), hp_f,
        prm.reshape(8 * 128))

    out = _epilogue(partials.reshape(4, 128), prm)
    return out.reshape(())


# speedup vs baseline: 3.5604x; 1.8045x over previous
import functools

import jax
import jax.numpy as jnp
from jax import lax
from jax.experimental import pallas as pl
from jax.experimental.pallas import tpu as pltpu
from jax.experimental.pallas import tpu_sc as plsc

N = 4096
NC = 2
NS = 16
NW = NC * NS
L = 16
RPW = N // NW
NCHUNK = N // L
RB = 16


def _prologue_body(sim_ref, p_ref, hp_ref, iou_ref, hn_ref, w_ref, qn_ref,
                   prm_ref):
    p = p_ref[...]
    hp = hp_ref[...]
    iou = iou_ref[...]
    sim = sim_ref[...]
    hn = p + hp - 2.0 * p * hp
    S = 1.0 + jnp.sum(hn * iou)
    logS = jnp.log(S)
    w = jnp.where(hn > 0.0, jnp.log(iou) - logS, 0.0)
    qn = jnp.exp(sim) * hp
    T = jnp.sum(qn)
    cnt = jnp.sum(hp)
    hn_ref[...] = hn
    w_ref[...] = w
    qn_ref[...] = qn
    z = jnp.float32(0.0)
    vals = jnp.stack([-logS, T, cnt, z, z, z, z, z])
    prm_ref[...] = jnp.broadcast_to(vals[:, None], (8, 128))


_prologue = pl.pallas_call(
    _prologue_body,
    out_shape=(
        jax.ShapeDtypeStruct((32, 128), jnp.float32),
        jax.ShapeDtypeStruct((32, 128), jnp.float32),
        jax.ShapeDtypeStruct((32, 128), jnp.float32),
        jax.ShapeDtypeStruct((8, 128), jnp.float32),
    ),
)


def _sc_body(d_hbm, hn_hbm, w_hbm, qn_hbm, hp_hbm, prm_hbm, out_hbm,
             hn_v, w_v, qn_v, hploc_v, cidx_v, wc_v, ridx_v, rows_v,
             mlogs_v, t_v, tot_v):
    c = lax.axis_index("c")
    s = lax.axis_index("s")
    wid = s * NC + c
    row0 = wid * RPW

    pltpu.sync_copy(hn_hbm, hn_v)
    pltpu.sync_copy(w_hbm, w_v)
    pltpu.sync_copy(qn_hbm, qn_v)
    pltpu.sync_copy(hp_hbm.at[pl.ds(row0, RPW)], hploc_v.at[pl.ds(0, RPW)])
    pltpu.sync_copy(prm_hbm.at[pl.ds(0, L)], mlogs_v)
    pltpu.sync_copy(prm_hbm.at[pl.ds(128, L)], t_v)
    mlogs = mlogs_v[...]
    t16 = t_v[...]

    iota = lax.iota(jnp.int32, L)
    izero = jnp.zeros((L,), jnp.int32)
    fzero = jnp.zeros((L,), jnp.float32)

    def col_cmp(j, cnt):
        base = j * L
        hn16 = hn_v[pl.ds(base, L)]
        m = hn16 > 0.0
        mi = m.astype(jnp.int32)
        offs = jnp.full((L,), cnt, jnp.int32) + plsc.cumsum(mi) - mi
        plsc.store_scatter(cidx_v, [offs], base + iota, mask=m)
        plsc.store_scatter(wc_v, [offs], w_v[pl.ds(base, L)], mask=m)
        return cnt + jnp.sum(mi)

    cnt_hn = lax.fori_loop(0, NCHUNK, col_cmp, jnp.int32(0))
    plsc.store_scatter(cidx_v, [cnt_hn + iota], izero, mask=None)
    plsc.store_scatter(wc_v, [cnt_hn + iota], fzero, mask=None)
    nchunk_c = (cnt_hn + L - 1) // L
    npad = nchunk_c * L - cnt_hn
    npad_f = jnp.full((L,), npad, jnp.int32).astype(jnp.float32)

    def row_cmp(j, cnt):
        base = j * L
        hp16 = hploc_v[pl.ds(base, L)]
        m = hp16 > 0.0
        mi = m.astype(jnp.int32)
        offs = jnp.full((L,), cnt, jnp.int32) + plsc.cumsum(mi) - mi
        plsc.store_scatter(ridx_v, [offs], row0 + base + iota, mask=m)
        return cnt + jnp.sum(mi)

    my_cnt = lax.fori_loop(0, RPW // L, row_cmp, jnp.int32(0))
    plsc.store_scatter(ridx_v, [my_cnt + iota],
                       jnp.full((L,), row0, jnp.int32), mask=None)
    nblk = (my_cnt + RB - 1) // RB

    def blk(g, total):
        pltpu.sync_copy(d_hbm.at[ridx_v.at[pl.ds(g * RB, RB)]], rows_v)

        def chunk(j, carry):
            base = j * L
            idx16 = cidx_v[pl.ds(base, L)]
            w16 = wc_v[pl.ds(base, L)]
            outs = []
            for r in range(RB):
                e = jnp.exp(
                    plsc.load_gather(rows_v,
                                     [jnp.full((L,), r, jnp.int32), idx16]))
                outs.append(carry[2 * r] + e)
                outs.append(carry[2 * r + 1] + e * w16)
            return tuple(outs)

        accs = lax.fori_loop(0, nchunk_c, chunk, (fzero,) * (2 * RB))

        e0 = jnp.exp(plsc.load_gather(rows_v, [iota, izero]))
        ra = fzero
        rb = fzero
        for r in range(RB):
            sel = iota == r
            ra = jnp.where(sel, jnp.full((L,), jnp.sum(accs[2 * r])), ra)
            rb = jnp.where(sel, jnp.full((L,), jnp.sum(accs[2 * r + 1])), rb)
        ra = ra - npad_f * e0

        ridx16 = ridx_v[pl.ds(g * RB, L)]
        qn16 = plsc.load_gather(qn_v, [ridx16])
        valid = (g * RB + iota) < my_cnt
        contrib = (qn16 * mlogs + rb) / (ra + t16)
        return total + jnp.where(valid, contrib, fzero)

    total = lax.fori_loop(0, nblk, blk, fzero)

    tot_v[...] = total
    pltpu.sync_copy(tot_v, out_hbm.at[pl.ds(wid * L, L)])


_sc_main = functools.partial(
    pl.kernel,
    out_type=jax.ShapeDtypeStruct((NW * L,), jnp.float32),
    mesh=plsc.VectorSubcoreMesh(core_axis_name="c", subcore_axis_name="s"),
    compiler_params=pltpu.CompilerParams(needs_layout_passes=False),
    scratch_types=[
        pltpu.VMEM((N,), jnp.float32),
        pltpu.VMEM((N,), jnp.float32),
        pltpu.VMEM((N,), jnp.float32),
        pltpu.VMEM((RPW + L,), jnp.float32),
        pltpu.VMEM((N + L,), jnp.int32),
        pltpu.VMEM((N + L,), jnp.float32),
        pltpu.VMEM((RPW + L,), jnp.int32),
        pltpu.VMEM((RB, N), jnp.float32),
        pltpu.VMEM((L,), jnp.float32),
        pltpu.VMEM((L,), jnp.float32),
        pltpu.VMEM((L,), jnp.float32),
    ],
)(_sc_body)


def _epilogue_body(part_ref, prm_ref, out_ref):
    loss = -jnp.sum(part_ref[...])
    cnt = prm_ref[...][2, 0]
    out_ref[...] = jnp.where(cnt == 0.0, 0.0, loss)[None, None]


_epilogue = pl.pallas_call(
    _epilogue_body,
    out_shape=jax.ShapeDtypeStruct((1, 1), jnp.float32),
)


def kernel(sim_mat, database_sim_mat, pos_mask, hard_pos_mask, neg_mask, iou):
    del neg_mask
    p_f = pos_mask[:, 1].astype(jnp.float32)
    hp_f = hard_pos_mask[:, 1].astype(jnp.float32)
    iou_ = iou[:, 0]

    hn, w, qn, prm = _prologue(
        sim_mat.reshape(32, 128), p_f.reshape(32, 128),
        hp_f.reshape(32, 128), iou_.reshape(32, 128))

    partials = _sc_main(
        database_sim_mat,
        hn.reshape(N), w.reshape(N), qn.reshape(N), hp_f,
        prm.reshape(8 * 128))

    out = _epilogue(partials.reshape(4, 128), prm)
    return out.reshape(())


# speedup vs baseline: 4.4182x; 1.2409x over previous
import functools

import jax
import jax.numpy as jnp
from jax import lax
from jax.experimental import pallas as pl
from jax.experimental.pallas import tpu as pltpu
from jax.experimental.pallas import tpu_sc as plsc

N = 4096
NC = 2
NS = 16
NW = NC * NS
L = 16
RPW = N // NW
NCHUNK = N // L
RB = 8


def _prologue_body(sim_ref, p_ref, hp_ref, iou_ref, hn_ref, w_ref, qn_ref,
                   prm_ref):
    p = p_ref[...]
    hp = hp_ref[...]
    iou = iou_ref[...]
    sim = sim_ref[...]
    hn = p + hp - 2.0 * p * hp
    S = 1.0 + jnp.sum(hn * iou)
    logS = jnp.log(S)
    w = jnp.where(hn > 0.0, jnp.log(iou) - logS, 0.0)
    qn = jnp.exp(sim) * hp
    T = jnp.sum(qn)
    cnt = jnp.sum(hp)
    hn_ref[...] = hn
    w_ref[...] = w
    qn_ref[...] = qn
    z = jnp.float32(0.0)
    vals = jnp.stack([-logS, T, cnt, z, z, z, z, z])
    prm_ref[...] = jnp.broadcast_to(vals[:, None], (8, 128))


_prologue = pl.pallas_call(
    _prologue_body,
    out_shape=(
        jax.ShapeDtypeStruct((32, 128), jnp.float32),
        jax.ShapeDtypeStruct((32, 128), jnp.float32),
        jax.ShapeDtypeStruct((32, 128), jnp.float32),
        jax.ShapeDtypeStruct((8, 128), jnp.float32),
    ),
)


def _sc_body(d_hbm, hn_hbm, w_hbm, qn_hbm, hp_hbm, prm_hbm, out_hbm,
             hn_v, w_v, qn_v, hploc_v, cidx_v, wc_v, ridx_v, rows_v,
             mlogs_v, t_v, tot_v, sem0, sem1):
    c = lax.axis_index("c")
    s = lax.axis_index("s")
    wid = s * NC + c
    row0 = wid * RPW

    pltpu.sync_copy(hn_hbm, hn_v)
    pltpu.sync_copy(w_hbm, w_v)
    pltpu.sync_copy(qn_hbm, qn_v)
    pltpu.sync_copy(hp_hbm.at[pl.ds(row0, RPW)], hploc_v.at[pl.ds(0, RPW)])
    pltpu.sync_copy(prm_hbm.at[pl.ds(0, L)], mlogs_v)
    pltpu.sync_copy(prm_hbm.at[pl.ds(128, L)], t_v)
    mlogs = mlogs_v[...]
    t16 = t_v[...]

    iota = lax.iota(jnp.int32, L)
    izero = jnp.zeros((L,), jnp.int32)
    fzero = jnp.zeros((L,), jnp.float32)

    def col_cmp(j, cnt):
        base = j * L
        hn16 = hn_v[pl.ds(base, L)]
        m = hn16 > 0.0
        mi = m.astype(jnp.int32)
        offs = jnp.full((L,), cnt, jnp.int32) + plsc.cumsum(mi) - mi
        plsc.store_scatter(cidx_v, [offs], base + iota, mask=m)
        plsc.store_scatter(wc_v, [offs], w_v[pl.ds(base, L)], mask=m)
        return cnt + jnp.sum(mi)

    cnt_hn = lax.fori_loop(0, NCHUNK, col_cmp, jnp.int32(0))
    plsc.store_scatter(cidx_v, [cnt_hn + iota], izero, mask=None)
    plsc.store_scatter(wc_v, [cnt_hn + iota], fzero, mask=None)
    nchunk_c = (cnt_hn + L - 1) // L
    npad = nchunk_c * L - cnt_hn
    npad_f = jnp.full((L,), npad, jnp.int32).astype(jnp.float32)

    def row_cmp(j, cnt):
        base = j * L
        hp16 = hploc_v[pl.ds(base, L)]
        m = hp16 > 0.0
        mi = m.astype(jnp.int32)
        offs = jnp.full((L,), cnt, jnp.int32) + plsc.cumsum(mi) - mi
        plsc.store_scatter(ridx_v, [offs], row0 + base + iota, mask=m)
        return cnt + jnp.sum(mi)

    my_cnt = lax.fori_loop(0, RPW // L, row_cmp, jnp.int32(0))
    plsc.store_scatter(ridx_v, [my_cnt + iota],
                       jnp.full((L,), row0, jnp.int32), mask=None)
    nblk = (my_cnt + RB - 1) // RB

    def issue(g):
        idxs = ridx_v.at[pl.ds(g * RB, RB)]

        @pl.when(lax.rem(g, 2) == 0)
        def _():
            pltpu.async_copy(d_hbm.at[idxs], rows_v.at[pl.ds(0, RB)], sem0)

        @pl.when(lax.rem(g, 2) == 1)
        def _():
            pltpu.async_copy(d_hbm.at[idxs], rows_v.at[pl.ds(RB, RB)], sem1)

    @pl.when(nblk > 0)
    def _():
        issue(0)

    def blk(g, total):
        par = lax.rem(g, 2)

        @pl.when(g + 1 < nblk)
        def _():
            issue(g + 1)

        @pl.when(par == 0)
        def _():
            pltpu.make_async_copy(
                d_hbm.at[ridx_v.at[pl.ds(0, RB)]],
                rows_v.at[pl.ds(0, RB)], sem0).wait()

        @pl.when(par == 1)
        def _():
            pltpu.make_async_copy(
                d_hbm.at[ridx_v.at[pl.ds(0, RB)]],
                rows_v.at[pl.ds(RB, RB)], sem1).wait()

        rbase = par * RB

        def chunk(j, carry):
            base = j * L
            idx16 = cidx_v[pl.ds(base, L)]
            w16 = wc_v[pl.ds(base, L)]
            outs = []
            for r in range(RB):
                e = jnp.exp(
                    plsc.load_gather(
                        rows_v,
                        [jnp.full((L,), rbase + r, jnp.int32), idx16]))
                outs.append(carry[2 * r] + e)
                outs.append(carry[2 * r + 1] + e * w16)
            return tuple(outs)

        accs = lax.fori_loop(0, nchunk_c, chunk, (fzero,) * (2 * RB))

        e0 = jnp.exp(plsc.load_gather(
            rows_v, [jnp.minimum(iota, RB - 1) + rbase, izero]))
        ra = fzero
        rb = fzero
        for r in range(RB):
            sel = iota == r
            ra = jnp.where(sel, jnp.full((L,), jnp.sum(accs[2 * r])), ra)
            rb = jnp.where(sel, jnp.full((L,), jnp.sum(accs[2 * r + 1])), rb)
        ra = ra - npad_f * e0

        ridx16 = ridx_v[pl.ds(g * RB, L)]
        qn16 = plsc.load_gather(qn_v, [ridx16])
        valid = ((g * RB + iota) < my_cnt) & (iota < RB)
        contrib = (qn16 * mlogs + rb) / (ra + t16)
        return total + jnp.where(valid, contrib, fzero)

    total = lax.fori_loop(0, nblk, blk, fzero)

    tot_v[...] = total
    pltpu.sync_copy(tot_v, out_hbm.at[pl.ds(wid * L, L)])


_sc_main = functools.partial(
    pl.kernel,
    out_type=jax.ShapeDtypeStruct((NW * L,), jnp.float32),
    mesh=plsc.VectorSubcoreMesh(core_axis_name="c", subcore_axis_name="s"),
    compiler_params=pltpu.CompilerParams(needs_layout_passes=False),
    scratch_types=[
        pltpu.VMEM((N,), jnp.float32),
        pltpu.VMEM((N,), jnp.float32),
        pltpu.VMEM((N,), jnp.float32),
        pltpu.VMEM((RPW + L,), jnp.float32),
        pltpu.VMEM((N + L,), jnp.int32),
        pltpu.VMEM((N + L,), jnp.float32),
        pltpu.VMEM((RPW + L,), jnp.int32),
        pltpu.VMEM((2 * RB, N), jnp.float32),
        pltpu.VMEM((L,), jnp.float32),
        pltpu.VMEM((L,), jnp.float32),
        pltpu.VMEM((L,), jnp.float32),
        pltpu.SemaphoreType.DMA,
        pltpu.SemaphoreType.DMA,
    ],
)(_sc_body)


def _epilogue_body(part_ref, prm_ref, out_ref):
    loss = -jnp.sum(part_ref[...])
    cnt = prm_ref[...][2, 0]
    out_ref[...] = jnp.where(cnt == 0.0, 0.0, loss)[None, None]


_epilogue = pl.pallas_call(
    _epilogue_body,
    out_shape=jax.ShapeDtypeStruct((1, 1), jnp.float32),
)


def kernel(sim_mat, database_sim_mat, pos_mask, hard_pos_mask, neg_mask, iou):
    del neg_mask
    p_f = pos_mask[:, 1].astype(jnp.float32)
    hp_f = hard_pos_mask[:, 1].astype(jnp.float32)
    iou_ = iou[:, 0]

    hn, w, qn, prm = _prologue(
        sim_mat.reshape(32, 128), p_f.reshape(32, 128),
        hp_f.reshape(32, 128), iou_.reshape(32, 128))

    partials = _sc_main(
        database_sim_mat,
        hn.reshape(N), w.reshape(N), qn.reshape(N), hp_f,
        prm.reshape(8 * 128))

    out = _epilogue(partials.reshape(4, 128), prm)
    return out.reshape(())
